# baseline (speedup 1.0000x reference)
import functools
import jax
import jax.numpy as jnp
from jax import lax
from jax.experimental import pallas as pl
from jax.experimental.pallas import tpu as pltpu
from jax.experimental.pallas import tpu_sc as plsc

EPS = 1e-6
L = 16
NC, NS = 2, 16
NW = NC * NS


_GATHER_DN = lax.GatherDimensionNumbers(
    offset_dims=(), collapsed_slice_dims=(0,), start_index_map=(0,))


def _lane_perm(v, idx):
    return lax.gather(v, idx[:, None], _GATHER_DN, slice_sizes=(1,),
                      mode=lax.GatherScatterMode.PROMISE_IN_BOUNDS)


def _rsqrt_vec(x):
    i = lax.bitcast_convert_type(x, jnp.int32)
    i = jnp.int32(0x5F3759DF) - lax.shift_right_arithmetic(i, jnp.int32(1))
    y = lax.bitcast_convert_type(i, jnp.float32)
    half = x * 0.5
    for _ in range(3):
        y = y * (1.5 - half * y * y)
    return y


def _make_sc_kernel(B, T, H, G, UN):
    NB = 4
    rows = T // NW
    ng = rows // G
    ch = H // L
    mesh = plsc.VectorSubcoreMesh(core_axis_name="c", subcore_axis_name="s")

    @functools.partial(
        pl.kernel,
        out_type=jax.ShapeDtypeStruct((B, T, H), jnp.float32),
        mesh=mesh,
        compiler_params=pltpu.CompilerParams(needs_layout_passes=False),
        scratch_types=[
            pltpu.VMEM((NB, G, H), jnp.float32),
            pltpu.VMEM((NB, B, G, H), jnp.float32),
            pltpu.VMEM((H,), jnp.float32),
            pltpu.VMEM((H,), jnp.float32),
            pltpu.SemaphoreType.DMA((NB,)),
            pltpu.SemaphoreType.DMA((NB,)),
            pltpu.SemaphoreType.DMA((NB,)),
        ],
    )
    def sc_kernel(x_hbm, tab_hbm, gamma_hbm, beta_hbm, out_hbm,
                  tab_v, io_v, g_v, b_v, tsem, isem, ssem):
        wid = lax.axis_index("s") * NC + lax.axis_index("c")
        base = wid * rows
        riota = lax.iota(jnp.int32, L)
        perms = [lax.rem(riota + k, jnp.int32(L)) for k in (8, 4, 2, 1)]
        zeros = jnp.zeros((L,), jnp.float32)

        def tab_cp(g, par):
            row0 = base + g * G
            return pltpu.make_async_copy(
                tab_hbm.at[pl.ds(row0, G), :], tab_v.at[par], tsem.at[par])

        def in_cps(g, par):
            row0 = base + g * G
            return [pltpu.make_async_copy(
                x_hbm.at[:, pl.ds(row0, G), :], io_v.at[par],
                isem.at[par])]

        def load_cps(g, par):
            return [tab_cp(g, par)] + in_cps(g, par)

        def store_cps(g, par):
            row0 = base + g * G
            return [pltpu.make_async_copy(
                io_v.at[par], out_hbm.at[:, pl.ds(row0, G), :],
                ssem.at[par])]

        for gg in range(min(2, ng)):
            for c in load_cps(gg, gg):
                c.start()
        pltpu.sync_copy(gamma_hbm, g_v)
        pltpu.sync_copy(beta_hbm, b_v)

        def group_body(g, carry):
            par = lax.rem(g, NB)
            pf = lax.rem(g + 2, NB)

            @pl.when(g >= 2)
            def _drain_prev_stores():
                for c in store_cps(g - 2, pf):
                    c.wait()

            @pl.when(g + 2 < ng)
            def _prefetch():
                for c in load_cps(g + 2, pf):
                    c.start()

            tab_cp(g, par).wait()

            rs = []
            mrs = []
            for r in range(G):
                @plsc.parallel_loop(0, ch, 4, unroll=2, carry=(zeros,) * 8)
                def row_stats(c, cr):
                    out = list(cr)
                    for k in range(4):
                        v = tab_v[par, r, pl.ds((c + k) * L, L)]
                        out[k] = out[k] + v
                        out[4 + k] = out[4 + k] + v * v
                    return tuple(out)
                acc = row_stats[0] + row_stats[1] + row_stats[2] + row_stats[3]
                acc2 = row_stats[4] + row_stats[5] + row_stats[6] + row_stats[7]
                for p in perms:
                    acc = acc + _lane_perm(acc, p)
                    acc2 = acc2 + _lane_perm(acc2, p)
                mean_r = acc * (1.0 / H)
                var_r = acc2 * (1.0 / H) - mean_r * mean_r
                rstd_r = _rsqrt_vec(var_r + EPS)
                rs.append(rstd_r[0])
                mrs.append(mean_r[0] * rs[r])

            for c in in_cps(g, par):
                c.wait()

            @plsc.parallel_loop(0, ch, 1, unroll=UN)
            def f_body(c):
                sl = pl.ds(c * L, L)
                gv = g_v[sl]
                bv = b_v[sl]
                ts = [(tab_v[par, r, sl] * rs[r] - mrs[r]) * gv + bv
                      for r in range(G)]
                for r in range(G):
                    for b in range(B):
                        plsc.addupdate(io_v.at[par, b, r, sl], ts[r])

            for c in store_cps(g, par):
                c.start()
            return carry

        lax.fori_loop(0, ng, group_body, 0)
        for gg in range(max(0, ng - 2), ng):
            for c in store_cps(gg, gg % NB):
                c.wait()

    return sc_kernel


def kernel(inputs, table, gamma, beta, dimensions):
    B, T, H = inputs.shape
    sc = _make_sc_kernel(B, T, H, G=4, UN=8)
    return sc(inputs, table, gamma, beta)


# speedup vs baseline: 1.0192x; 1.0192x over previous
import functools
import jax
import jax.numpy as jnp
from jax import lax
from jax.experimental import pallas as pl
from jax.experimental.pallas import tpu as pltpu
from jax.experimental.pallas import tpu_sc as plsc

EPS = 1e-6
L = 16
NC, NS = 2, 16
NW = NC * NS


_GATHER_DN = lax.GatherDimensionNumbers(
    offset_dims=(), collapsed_slice_dims=(0,), start_index_map=(0,))


def _lane_perm(v, idx):
    return lax.gather(v, idx[:, None], _GATHER_DN, slice_sizes=(1,),
                      mode=lax.GatherScatterMode.PROMISE_IN_BOUNDS)


def _rsqrt_vec(x):
    i = lax.bitcast_convert_type(x, jnp.int32)
    i = jnp.int32(0x5F3759DF) - lax.shift_right_arithmetic(i, jnp.int32(1))
    y = lax.bitcast_convert_type(i, jnp.float32)
    half = x * 0.5
    for _ in range(3):
        y = y * (1.5 - half * y * y)
    return y


def _make_sc_kernel(B, T, H, G, UN):
    NB = 4
    rows = T // NW
    ng = rows // G
    ch = H // L
    mesh = plsc.VectorSubcoreMesh(core_axis_name="c", subcore_axis_name="s")

    @functools.partial(
        pl.kernel,
        out_type=jax.ShapeDtypeStruct((B, T, H), jnp.float32),
        mesh=mesh,
        compiler_params=pltpu.CompilerParams(needs_layout_passes=False),
        scratch_types=[
            pltpu.VMEM((NB, G, H), jnp.float32),
            pltpu.VMEM((NB, B, G, H), jnp.float32),
            pltpu.VMEM((H,), jnp.float32),
            pltpu.VMEM((H,), jnp.float32),
            pltpu.VMEM((2, G, L), jnp.float32),
            pltpu.SemaphoreType.DMA((NB,)),
            pltpu.SemaphoreType.DMA((NB,)),
            pltpu.SemaphoreType.DMA((NB,)),
        ],
    )
    def sc_kernel(x_hbm, tab_hbm, gamma_hbm, beta_hbm, out_hbm,
                  tab_v, io_v, g_v, b_v, st_v, tsem, isem, ssem):
        wid = lax.axis_index("s") * NC + lax.axis_index("c")
        base = wid * rows
        riota = lax.iota(jnp.int32, L)
        perms = [lax.rem(riota + k, jnp.int32(L)) for k in (8, 4, 2, 1)]
        zeros = jnp.zeros((L,), jnp.float32)

        def tab_cp(g, par):
            row0 = base + g * G
            return pltpu.make_async_copy(
                tab_hbm.at[pl.ds(row0, G), :], tab_v.at[par], tsem.at[par])

        def in_cps(g, par):
            row0 = base + g * G
            return [pltpu.make_async_copy(
                x_hbm.at[:, pl.ds(row0, G), :], io_v.at[par],
                isem.at[par])]

        def load_cps(g, par):
            return [tab_cp(g, par)] + in_cps(g, par)

        def store_cps(g, par):
            row0 = base + g * G
            return [pltpu.make_async_copy(
                io_v.at[par], out_hbm.at[:, pl.ds(row0, G), :],
                ssem.at[par])]

        for gg in range(min(2, ng)):
            for c in load_cps(gg, gg):
                c.start()
        pltpu.sync_copy(gamma_hbm, g_v)
        pltpu.sync_copy(beta_hbm, b_v)

        def group_body(g, carry):
            par = lax.rem(g, NB)
            pf = lax.rem(g + 2, NB)

            @pl.when(g >= 2)
            def _drain_prev_stores():
                for c in store_cps(g - 2, pf):
                    c.wait()

            @pl.when(g + 2 < ng)
            def _prefetch():
                for c in load_cps(g + 2, pf):
                    c.start()

            tab_cp(g, par).wait()

            def row_body(r, cr0):
                @plsc.parallel_loop(0, ch, 4, unroll=2, carry=(zeros,) * 8)
                def row_stats(c, cr):
                    out = list(cr)
                    for k in range(4):
                        v = tab_v[par, r, pl.ds((c + k) * L, L)]
                        out[k] = out[k] + v
                        out[4 + k] = out[4 + k] + v * v
                    return tuple(out)
                acc = row_stats[0] + row_stats[1] + row_stats[2] + row_stats[3]
                acc2 = row_stats[4] + row_stats[5] + row_stats[6] + row_stats[7]
                for p in perms:
                    acc = acc + _lane_perm(acc, p)
                    acc2 = acc2 + _lane_perm(acc2, p)
                mean_r = acc * (1.0 / H)
                var_r = acc2 * (1.0 / H) - mean_r * mean_r
                rstd_r = _rsqrt_vec(var_r + EPS)
                st_v[0, r, :] = rstd_r
                st_v[1, r, :] = mean_r * rstd_r
                return cr0
            lax.fori_loop(0, G, row_body, 0)
            rs = [st_v[0, r, :][0] for r in range(G)]
            mrs = [st_v[1, r, :][0] for r in range(G)]

            for c in in_cps(g, par):
                c.wait()

            @plsc.parallel_loop(0, ch, 1, unroll=UN)
            def f_body(c):
                sl = pl.ds(c * L, L)
                gv = g_v[sl]
                bv = b_v[sl]
                ts = [(tab_v[par, r, sl] * rs[r] - mrs[r]) * gv + bv
                      for r in range(G)]
                for r in range(G):
                    for b in range(B):
                        plsc.addupdate(io_v.at[par, b, r, sl], ts[r])

            for c in store_cps(g, par):
                c.start()
            return carry

        lax.fori_loop(0, ng, group_body, 0)
        for gg in range(max(0, ng - 2), ng):
            for c in store_cps(gg, gg % NB):
                c.wait()

    return sc_kernel


def kernel(inputs, table, gamma, beta, dimensions):
    B, T, H = inputs.shape
    sc = _make_sc_kernel(B, T, H, G=4, UN=4)
    return sc(inputs, table, gamma, beta)
